# baseline (speedup 1.0000x reference)
import functools

import jax
import jax.numpy as jnp
from jax import lax
from jax.experimental import pallas as pl
from jax.experimental.pallas import tpu as pltpu
from jax.experimental.pallas import tpu_sc as plsc

_NC = 2
_NS = 16
_NW = _NC * _NS
_LANES = 16

_CHUNK = 64
_DEPTH = 4
_AHEAD = 2


def _table_body(wt_ref, ct_ref, w_ref, out_ref, *, n_proj_blocks):
    i = pl.program_id(0)

    @pl.when(i < n_proj_blocks)
    def _():
        out_ref[...] = lax.dot_general(
            wt_ref[...], w_ref[...], (((1,), (1,)), ((), ())),
            preferred_element_type=jnp.float32)

    @pl.when(i >= n_proj_blocks)
    def _():
        out_ref[...] = ct_ref[...]


def _build_table(word_table, char_table, W_word):
    V, D = word_table.shape
    C = char_table.shape[0]
    E = W_word.shape[0]
    R = 10000
    assert V % R == 0 and C % R == 0
    npb = V // R
    ncb = C // R
    return pl.pallas_call(
        functools.partial(_table_body, n_proj_blocks=npb),
        grid=(npb + ncb,),
        in_specs=[
            pl.BlockSpec((R, D), lambda i: (jnp.minimum(i, npb - 1), 0)),
            pl.BlockSpec((R, D), lambda i: (jnp.maximum(i - npb, 0), 0)),
            pl.BlockSpec((E, D), lambda i: (0, 0)),
        ],
        out_specs=pl.BlockSpec((R, E), lambda i: (i, 0)),
        out_shape=jax.ShapeDtypeStruct((V + C, E), jnp.float32),
    )(word_table, char_table, W_word)


def _sc_mix_body(tab_hbm, idx_hbm, out_hbm, idx_v, rows_v,
                 sem_idx, sem_g0, sem_g1, sem_g2, sem_g3,
                 sem_s0, sem_s1, sem_s2, sem_s3):
    n_tok = out_hbm.shape[0]
    per_w = n_tok // _NW
    n_chunks = per_w // _CHUNK
    wid = lax.axis_index("s") * _NC + lax.axis_index("c")
    base = wid * per_w
    sem_g = (sem_g0, sem_g1, sem_g2, sem_g3)
    sem_s = (sem_s0, sem_s1, sem_s2, sem_s3)

    pltpu.async_copy(idx_hbm.at[wid], idx_v, sem_idx).wait()

    def gather_copy(g, slot):
        ii = idx_v.at[pl.ds(g * 2 * _CHUNK, 2 * _CHUNK)]
        return pltpu.make_async_copy(tab_hbm.at[ii], rows_v.at[slot],
                                     sem_g[slot])

    def out_copy(g, slot):
        off = pl.multiple_of(base + g * _CHUNK, _CHUNK)
        return pltpu.make_async_copy(rows_v.at[slot, pl.ds(0, _CHUNK)],
                                     out_hbm.at[pl.ds(off, _CHUNK)], sem_s[slot])

    for g0 in range(_AHEAD):
        gather_copy(g0, g0 % _DEPTH).start()

    def outer(o, carry):
        for b in range(_DEPTH):
            g = _DEPTH * o + b
            sa = (b + _AHEAD) % _DEPTH
            @pl.when(g + _AHEAD < n_chunks)
            def _():
                @pl.when(g + _AHEAD >= _DEPTH)
                def _():
                    out_copy(g + _AHEAD - _DEPTH, sa).wait()
                gather_copy(g + _AHEAD, sa).start()

            gather_copy(g, b).wait()

            def add_row(r, c2):
                for j in range(8):
                    sl = pl.ds(j * _LANES, _LANES)
                    rows_v[b, r, sl] = (rows_v[b, r, sl]
                                        + rows_v[b, _CHUNK + r, sl])
                return c2

            lax.fori_loop(0, _CHUNK, add_row, 0, unroll=False)
            out_copy(g, b).start()
        return carry

    lax.fori_loop(0, n_chunks // _DEPTH, outer, 0, unroll=False)
    for g0 in range(n_chunks - _DEPTH, n_chunks):
        out_copy(g0, g0 % _DEPTH).wait()


def _sc_mix(table, idx, n_tok):
    E = table.shape[1]
    per_w = n_tok // _NW
    return pl.kernel(
        _sc_mix_body,
        out_type=jax.ShapeDtypeStruct((n_tok, E), jnp.float32),
        mesh=plsc.VectorSubcoreMesh(core_axis_name="c", subcore_axis_name="s",
                                    num_cores=_NC, num_subcores=_NS),
        scratch_types=[
            pltpu.VMEM((2 * per_w,), jnp.int32),
            pltpu.VMEM((_DEPTH, 2 * _CHUNK, E), jnp.float32),
        ] + [pltpu.SemaphoreType.DMA] * 9,
    )(table, idx.reshape(_NW, 2 * per_w))


def kernel(word, char, word_table, char_table, W_word):
    B, L = word.shape
    V = word_table.shape[0]
    E = W_word.shape[0]
    n_tok = B * L
    per_w = n_tok // _NW
    n_chunks = per_w // _CHUNK

    table = _build_table(word_table, char_table, W_word)
    wi = word.reshape(-1).astype(jnp.int32).reshape(_NW, n_chunks, 1, _CHUNK)
    ci = (char.reshape(-1).astype(jnp.int32) + V).reshape(_NW, n_chunks, 1,
                                                          _CHUNK)
    idx = jnp.concatenate([wi, ci], axis=2).reshape(-1)
    out = _sc_mix(table, idx, n_tok)
    return out.reshape(B, L, E)


# speedup vs baseline: 1.1068x; 1.1068x over previous
import functools

import jax
import jax.numpy as jnp
from jax import lax
from jax.experimental import pallas as pl
from jax.experimental.pallas import tpu as pltpu
from jax.experimental.pallas import tpu_sc as plsc

_NC = 2
_NS = 16
_NW = _NC * _NS
_LANES = 16

_CHUNK = 32
_DEPTH = 8
_AHEAD = 4


def _proj_body(wt_ref, w_ref, out_ref):
    out_ref[...] = lax.dot_general(
        wt_ref[...], w_ref[...], (((1,), (1,)), ((), ())),
        preferred_element_type=jnp.float32)


def _project_table(word_table, W_word):
    V, D = word_table.shape
    E = W_word.shape[0]
    R = 20000
    assert V % R == 0
    return pl.pallas_call(
        _proj_body,
        grid=(V // R,),
        in_specs=[
            pl.BlockSpec((R, D), lambda i: (i, 0)),
            pl.BlockSpec((E, D), lambda i: (0, 0)),
        ],
        out_specs=pl.BlockSpec((R, E), lambda i: (i, 0)),
        out_shape=jax.ShapeDtypeStruct((V, E), jnp.float32),
    )(word_table, W_word)


def _sc_mix_body(proj_hbm, ctab_hbm, widx_hbm, cidx_hbm, out_hbm,
                 idxw_v, idxc_v, wrows_v, crows_v,
                 sem_idx, sem_g0, sem_g1, sem_g2, sem_g3, sem_g4, sem_g5,
                 sem_g6, sem_g7, sem_s0, sem_s1, sem_s2, sem_s3, sem_s4,
                 sem_s5, sem_s6, sem_s7):
    n_tok = out_hbm.shape[0]
    per_w = n_tok // _NW
    n_chunks = per_w // _CHUNK
    wid = lax.axis_index("s") * _NC + lax.axis_index("c")
    base = wid * per_w
    sem_g = (sem_g0, sem_g1, sem_g2, sem_g3, sem_g4, sem_g5, sem_g6,
             sem_g7)
    sem_s = (sem_s0, sem_s1, sem_s2, sem_s3, sem_s4, sem_s5, sem_s6,
             sem_s7)

    pltpu.async_copy(widx_hbm.at[wid], idxw_v, sem_idx)
    pltpu.async_copy(cidx_hbm.at[wid], idxc_v, sem_idx).wait()
    pltpu.make_async_copy(widx_hbm.at[wid], idxw_v, sem_idx).wait()

    def issue_gathers(g, slot):
        iw = idxw_v.at[pl.ds(g * _CHUNK, _CHUNK)]
        ic = idxc_v.at[pl.ds(g * _CHUNK, _CHUNK)]
        pltpu.async_copy(proj_hbm.at[iw], wrows_v.at[slot], sem_g[slot])
        pltpu.async_copy(ctab_hbm.at[ic], crows_v.at[slot], sem_g[slot])

    def wait_gathers(g, slot):
        iw = idxw_v.at[pl.ds(g * _CHUNK, _CHUNK)]
        ic = idxc_v.at[pl.ds(g * _CHUNK, _CHUNK)]
        pltpu.make_async_copy(proj_hbm.at[iw], wrows_v.at[slot],
                              sem_g[slot]).wait()
        pltpu.make_async_copy(ctab_hbm.at[ic], crows_v.at[slot],
                              sem_g[slot]).wait()

    def out_copy(g, slot):
        off = pl.multiple_of(base + g * _CHUNK, _CHUNK)
        return pltpu.make_async_copy(wrows_v.at[slot],
                                     out_hbm.at[pl.ds(off, _CHUNK)], sem_s[slot])

    for g0 in range(_AHEAD):
        issue_gathers(g0, g0 % _DEPTH)

    def outer(o, carry):
        for b in range(_DEPTH):
            g = _DEPTH * o + b
            sa = (b + _AHEAD) % _DEPTH
            @pl.when(g + _AHEAD < n_chunks)
            def _():
                @pl.when(g + _AHEAD >= _DEPTH)
                def _():
                    out_copy(g + _AHEAD - _DEPTH, sa).wait()
                issue_gathers(g + _AHEAD, sa)

            wait_gathers(g, b)

            def add_row(r, c2):
                for j in range(8):
                    sl = pl.ds(j * _LANES, _LANES)
                    wrows_v[b, r, sl] = wrows_v[b, r, sl] + crows_v[b, r, sl]
                return c2

            lax.fori_loop(0, _CHUNK, add_row, 0, unroll=False)
            out_copy(g, b).start()
        return carry

    lax.fori_loop(0, n_chunks // _DEPTH, outer, 0, unroll=False)
    for g0 in range(n_chunks - _DEPTH, n_chunks):
        out_copy(g0, g0 % _DEPTH).wait()


def _sc_mix(proj, char_table, widx, cidx):
    n_tok = widx.shape[0]
    E = proj.shape[1]
    per_w = n_tok // _NW
    return pl.kernel(
        _sc_mix_body,
        out_type=jax.ShapeDtypeStruct((n_tok, E), jnp.float32),
        mesh=plsc.VectorSubcoreMesh(core_axis_name="c", subcore_axis_name="s",
                                    num_cores=_NC, num_subcores=_NS),
        scratch_types=[
            pltpu.VMEM((per_w,), jnp.int32),
            pltpu.VMEM((per_w,), jnp.int32),
            pltpu.VMEM((_DEPTH, _CHUNK, E), jnp.float32),
            pltpu.VMEM((_DEPTH, _CHUNK, E), jnp.float32),
        ] + [pltpu.SemaphoreType.DMA] * 17,
    )(proj, char_table, widx.reshape(_NW, per_w), cidx.reshape(_NW, per_w))


def kernel(word, char, word_table, char_table, W_word):
    B, L = word.shape
    E = W_word.shape[0]
    proj = _project_table(word_table, W_word)
    widx = word.reshape(-1).astype(jnp.int32)
    cidx = char.reshape(-1).astype(jnp.int32)
    out = _sc_mix(proj, char_table, widx, cidx)
    return out.reshape(B, L, E)
